# baseline (speedup 1.0000x reference)
import numpy as np
import jax
import jax.numpy as jnp
from jax.experimental import pallas as pl

_RATIOS = (0.5, 1.0, 2.0)
_SCALES = (8, 16, 32)
_FEAT_STRIDE = 16
_BASE_SIZE = 16
_N_PRE = 12000
_N_POST = 2000
_NMS_TH = 0.7
_MIN_SIZE = 16.0


def _anchors(hh, ww):
    py = px = _BASE_SIZE / 2.0
    ab = np.zeros((9, 4), dtype=np.float32)
    for i, r in enumerate(_RATIOS):
        for j, s in enumerate(_SCALES):
            h = _BASE_SIZE * s * np.sqrt(r)
            w = _BASE_SIZE * s * np.sqrt(1.0 / r)
            k = i * 3 + j
            ab[k] = [py - h / 2, px - w / 2, py + h / 2, px + w / 2]
    sy = np.arange(0, hh * _FEAT_STRIDE, _FEAT_STRIDE)
    sx = np.arange(0, ww * _FEAT_STRIDE, _FEAT_STRIDE)
    sx, sy = np.meshgrid(sx, sy)
    shift = np.stack((sy.ravel(), sx.ravel(), sy.ravel(), sx.ravel()), axis=1)
    anchor = ab.reshape((1, 9, 4)) + shift.reshape((-1, 1, 4))
    return anchor.reshape((-1, 4)).astype(np.float32)


def _conv3x3(x, W1, b1):
    xh = jnp.transpose(x[0], (1, 2, 0))
    xp = jnp.pad(xh, ((1, 1), (1, 1), (0, 0)))
    xf = jnp.concatenate([xp.reshape(2704, 512), jnp.zeros((8, 512), jnp.float32)], axis=0)
    acc = jnp.zeros((2600, 512), jnp.float32)
    for ky in range(3):
        for kx in range(3):
            Wk = W1[:, :, ky, kx]
            tmp = jnp.dot(xf, Wk.T, preferred_element_type=jnp.float32)
            off = ky * 52 + kx
            acc = acc + jax.lax.dynamic_slice(tmp, (off, 0), (2600, 512))
    h = acc + b1[None, :]
    h = h.reshape(50, 52, 512)[:, :50, :].reshape(2500, 512)
    return jax.nn.relu(h)


def _nms_keep(boxes, valid, thresh):
    n = boxes.shape[0]
    areas = (boxes[:, 2] - boxes[:, 0]) * (boxes[:, 3] - boxes[:, 1])
    idxs = jnp.arange(n)

    def body(i, keep):
        yy1 = jnp.maximum(boxes[i, 0], boxes[:, 0])
        xx1 = jnp.maximum(boxes[i, 1], boxes[:, 1])
        yy2 = jnp.minimum(boxes[i, 2], boxes[:, 2])
        xx2 = jnp.minimum(boxes[i, 3], boxes[:, 3])
        inter = jnp.clip(yy2 - yy1, 0.0) * jnp.clip(xx2 - xx1, 0.0)
        iou = inter / (areas[i] + areas - inter + 1e-12)
        suppress = (iou > thresh) & (idxs > i) & keep[i]
        return keep & (~suppress)

    return jax.lax.fori_loop(0, n, body, valid)


def kernel(x, img_size, W1, b1, Wloc, bloc, Wscore, bscore):
    n, _, hh, ww = x.shape
    anchor = jnp.asarray(_anchors(hh, ww))
    h = _conv3x3(x, W1, b1)
    locs = jnp.dot(h, Wloc[:, :, 0, 0].T, preferred_element_type=jnp.float32) + bloc
    scores = jnp.dot(h, Wscore[:, :, 0, 0].T, preferred_element_type=jnp.float32) + bscore
    rpn_locs = locs.reshape(1, -1, 4)
    rpn_scores = scores.reshape(1, -1, 2)
    sm = jax.nn.softmax(scores.reshape(1, hh, ww, 9, 2), axis=4)
    fg = sm[:, :, :, :, 1].reshape(1, -1)

    def _noop(a_ref, o_ref):
        o_ref[...] = a_ref[...]
    fg = pl.pallas_call(_noop, out_shape=jax.ShapeDtypeStruct(fg.shape, fg.dtype))(fg)

    img_h = img_size[0].astype(jnp.float32)
    img_w = img_size[1].astype(jnp.float32)
    loc = rpn_locs[0]
    score = fg[0]
    src_h = anchor[:, 2] - anchor[:, 0]
    src_w = anchor[:, 3] - anchor[:, 1]
    src_cy = anchor[:, 0] + 0.5 * src_h
    src_cx = anchor[:, 1] + 0.5 * src_w
    dy, dx, dh, dw = loc[:, 0], loc[:, 1], loc[:, 2], loc[:, 3]
    cy = dy * src_h + src_cy
    cx = dx * src_w + src_cx
    bh = jnp.exp(dh) * src_h
    bw = jnp.exp(dw) * src_w
    roi = jnp.stack([cy - 0.5 * bh, cx - 0.5 * bw, cy + 0.5 * bh, cx + 0.5 * bw], axis=1)
    y1 = jnp.clip(roi[:, 0], 0.0, img_h)
    x1 = jnp.clip(roi[:, 1], 0.0, img_w)
    y2 = jnp.clip(roi[:, 2], 0.0, img_h)
    x2 = jnp.clip(roi[:, 3], 0.0, img_w)
    roi = jnp.stack([y1, x1, y2, x2], axis=1)
    valid = ((y2 - y1) >= _MIN_SIZE) & ((x2 - x1) >= _MIN_SIZE)
    sc = jnp.where(valid, score, -jnp.inf)
    order = jnp.argsort(-sc)[:_N_PRE]
    roi_o = roi[order]
    valid_o = jnp.isfinite(sc[order])
    keep = _nms_keep(roi_o, valid_o, _NMS_TH)
    idx = jnp.nonzero(keep, size=_N_POST, fill_value=0)[0]
    rois = roi_o[idx]
    roi_indices = jnp.zeros((_N_POST,), dtype=jnp.int32)
    return (rpn_locs, rpn_scores, rois, roi_indices, anchor)


# speedup vs baseline: 96.1484x; 96.1484x over previous
import functools
import numpy as np
import jax
import jax.numpy as jnp
from jax.experimental import pallas as pl

_RATIOS = (0.5, 1.0, 2.0)
_SCALES = (8, 16, 32)
_FEAT_STRIDE = 16
_BASE_SIZE = 16
_N_PRE = 12000
_N_POST = 2000
_NMS_TH = 0.7
_MIN_SIZE = 16.0

_HH = 50
_WW = 50
_HP = _HH + 2
_NQ = _HP * _HP
_NQV = _HH * _HP
_NSORT = 12288
_NT = _NSORT // 128


def _anchors_np(hh, ww):
    py = px = _BASE_SIZE / 2.0
    ab = np.zeros((9, 4), dtype=np.float32)
    for i, r in enumerate(_RATIOS):
        for j, s in enumerate(_SCALES):
            h = _BASE_SIZE * s * np.sqrt(r)
            w = _BASE_SIZE * s * np.sqrt(1.0 / r)
            ab[i * 3 + j] = [py - h / 2, px - w / 2, py + h / 2, px + w / 2]
    sy = np.arange(0, hh * _FEAT_STRIDE, _FEAT_STRIDE)
    sx = np.arange(0, ww * _FEAT_STRIDE, _FEAT_STRIDE)
    sx, sy = np.meshgrid(sx, sy)
    shift = np.stack((sy.ravel(), sx.ravel(), sy.ravel(), sx.ravel()), axis=1)
    anchor = ab.reshape((1, 9, 4)) + shift.reshape((-1, 1, 4))
    return anchor.reshape((-1, 4)).astype(np.float32)


def _anchor_planes_np(hh, ww):
    a = _anchors_np(hh, ww).reshape(hh, ww, 9, 4)
    planes = []
    for comp in range(4):
        p = np.ones((hh, _HP, 16), dtype=np.float32)
        p[:, :ww, :9] = a[:, :, :, comp]
        planes.append(p.reshape(_NQV, 16))
    ay1, ax1, ay2, ax2 = planes
    src_h = ay2 - ay1
    src_w = ax2 - ax1
    src_cy = ay1 + 0.5 * src_h
    src_cx = ax1 + 0.5 * src_w
    return src_h, src_w, src_cy, src_cx


def _conv_body(xf_ref, w1_ref, acc_ref):
    for k in range(9):
        off = (k // 3) * _HP + (k % 3)
        t = jnp.dot(xf_ref[off:off + _NQV, :].astype(jnp.bfloat16),
                    w1_ref[k].astype(jnp.bfloat16),
                    preferred_element_type=jnp.float32)
        if k == 0:
            acc_ref[...] = t
        else:
            acc_ref[...] += t


def _head_body(accr, b1, wh, bh, sh, sw, scy, scx, ihw,
               dy_o, dx_o, dh_o, dw_o, s0_o, s1_o, sc_o,
               y1_o, x1_o, y2_o, x2_o):
    h = jnp.maximum(accr[...] + b1[0:1, :], 0.0)
    hd = jnp.dot(h.astype(jnp.bfloat16), wh[...].astype(jnp.bfloat16),
                 preferred_element_type=jnp.float32) + bh[0:1, :]
    dy = hd[:, 0:16]
    dx = hd[:, 16:32]
    dh = hd[:, 32:48]
    dw = hd[:, 48:64]
    s0 = hd[:, 64:80]
    s1 = hd[:, 80:96]
    src_h = sh[...]
    src_w = sw[...]
    cy = dy * src_h + scy[...]
    cx = dx * src_w + scx[...]
    bhh = jnp.exp(dh) * src_h
    bww = jnp.exp(dw) * src_w
    img_h = ihw[0, 0]
    img_w = ihw[0, 1]
    y1 = jnp.clip(cy - 0.5 * bhh, 0.0, img_h)
    x1 = jnp.clip(cx - 0.5 * bww, 0.0, img_w)
    y2 = jnp.clip(cy + 0.5 * bhh, 0.0, img_h)
    x2 = jnp.clip(cx + 0.5 * bww, 0.0, img_w)
    m = jnp.maximum(s0, s1)
    e0 = jnp.exp(s0 - m)
    e1 = jnp.exp(s1 - m)
    fg = e1 / (e0 + e1)
    valid = ((y2 - y1) >= _MIN_SIZE) & ((x2 - x1) >= _MIN_SIZE)
    sc = jnp.where(valid, fg, -jnp.inf)
    dy_o[...] = dy
    dx_o[...] = dx
    dh_o[...] = dh
    dw_o[...] = dw
    s0_o[...] = s0
    s1_o[...] = s1
    sc_o[...] = sc
    y1_o[...] = y1
    x1_o[...] = x1
    y2_o[...] = y2
    x2_o[...] = x2


def _run_head(x, W1, b1, Wloc, bloc, Wscore, bscore, img_size):
    xh = jnp.transpose(x[0], (1, 2, 0))
    xp = jnp.pad(xh, ((1, 1), (1, 1), (0, 0)))
    xf = jnp.concatenate(
        [xp.reshape(_NQ, 512), jnp.zeros((8, 512), jnp.float32)], axis=0)
    w1 = jnp.transpose(W1, (2, 3, 1, 0)).reshape(9, 512, 512)
    Wl = Wloc[:, :, 0, 0]
    Ws = Wscore[:, :, 0, 0]
    cols = []
    bcols = []
    for g in range(4):
        cols.append(Wl[g::4].T)
        bcols.append(bloc[g::4])
    for g in range(2):
        cols.append(Ws[g::2].T)
        bcols.append(bscore[g::2])
    pad9 = lambda a: jnp.pad(a, ((0, 0), (0, 7)))
    wh = jnp.pad(jnp.concatenate([pad9(c) for c in cols], axis=1),
                 ((0, 0), (0, 32)))
    bh = jnp.pad(jnp.concatenate(
        [jnp.pad(b, (0, 7)) for b in bcols]), (0, 32)).reshape(1, 128)
    sh, sw, scy, scx = (jnp.asarray(p) for p in _anchor_planes_np(_HH, _WW))
    ihw = img_size.astype(jnp.float32).reshape(1, 2)
    po = jax.ShapeDtypeStruct((_NQV, 16), jnp.float32)
    from jax.experimental.pallas import tpu as pltpu
    acc = pl.pallas_call(
        _conv_body,
        out_shape=jax.ShapeDtypeStruct((_NQV, 512), jnp.float32),
    )(xf, w1)
    outs = pl.pallas_call(
        _head_body,
        out_shape=(po,) * 11,
    )(acc, b1.reshape(1, 512), wh, bh, sh, sw, scy, scx, ihw)
    return outs


def _outer_col(row):
    return jnp.transpose(row)


def _nms_body(y1r, x1r, y2r, x2r, vr, keep_o, keep_s, cnt_s):
    keep_s[...] = vr[...]
    cnt_s[0] = 0.0
    lane = jax.lax.broadcasted_iota(jnp.int32, (128, 128), 1)
    sub = jax.lax.broadcasted_iota(jnp.int32, (128, 128), 0)
    upper = lane > sub

    def load_rows(refs, t):
        return tuple(r[pl.ds(t, 1), :] for r in refs)

    refs = (y1r, x1r, y2r, x2r)

    def iou_mat(cols, rows):
        (y1c, x1c, y2c, x2c, ac) = cols
        y1u, x1u, y2u, x2u = rows
        au = (y2u - y1u) * (x2u - x1u)
        yy1 = jnp.maximum(y1c, y1u)
        xx1 = jnp.maximum(x1c, x1u)
        yy2 = jnp.minimum(y2c, y2u)
        xx2 = jnp.minimum(x2c, x2u)
        inter = jnp.maximum(yy2 - yy1, 0.0) * jnp.maximum(xx2 - xx1, 0.0)
        return inter / (ac + au - inter + 1e-12)

    def outer(t, _):
        do = cnt_s[0] < float(_N_POST)

        @pl.when(do)
        def _process():
            rows_t = load_rows(refs, t)
            (y1t, x1t, y2t, x2t) = rows_t
            at = (y2t - y1t) * (x2t - x1t)
            cols = tuple(_outer_col(r) for r in (y1t, x1t, y2t, x2t, at))
            smat = jnp.where((iou_mat(cols, rows_t) > _NMS_TH) & upper,
                             1.0, 0.0)
            v = keep_s[pl.ds(t, 1), :]

            def fix_cond(c):
                alive, dk = c
                return jnp.sum(alive - dk) > 0.0

            def fix_body(c):
                alive, dk = c
                dk2 = v * jnp.where(
                    jnp.dot(alive, smat,
                            preferred_element_type=jnp.float32) > 0.0,
                    0.0, 1.0)
                alive2 = alive * jnp.where(
                    jnp.dot(dk2, smat,
                            preferred_element_type=jnp.float32) > 0.0,
                    0.0, 1.0)
                return alive2, dk2

            alive, _dk = jax.lax.while_loop(
                fix_cond, fix_body, (v, jnp.zeros_like(v)))
            keep_s[pl.ds(t, 1), :] = alive
            cnt_s[0] = cnt_s[0] + jnp.sum(alive)

            def inner(u, carry):
                rows_u = load_rows(refs, u)
                sx = jnp.where(iou_mat(cols, rows_u) > _NMS_TH, 1.0, 0.0)
                press = jnp.dot(alive, sx,
                                preferred_element_type=jnp.float32)
                ku = keep_s[pl.ds(u, 1), :]
                keep_s[pl.ds(u, 1), :] = ku * jnp.where(press > 0.0, 0.0, 1.0)
                return carry

            jax.lax.fori_loop(t + 1, _NT, inner, 0)

        @pl.when(jnp.logical_not(do))
        def _skip():
            keep_s[pl.ds(t, 1), :] = jnp.zeros((1, 128), jnp.float32)

        return _

    jax.lax.fori_loop(0, _NT, outer, 0)
    keep_o[...] = keep_s[...]


def _run_nms(y1o, x1o, y2o, x2o, valid_o):
    from jax.experimental.pallas import tpu as pltpu
    po = jax.ShapeDtypeStruct((_NT, 128), jnp.float32)
    return pl.pallas_call(
        _nms_body,
        out_shape=po,
        scratch_shapes=[pltpu.VMEM((_NT, 128), jnp.float32),
                        pltpu.SMEM((1,), jnp.float32)],
    )(y1o.reshape(_NT, 128), x1o.reshape(_NT, 128),
      y2o.reshape(_NT, 128), x2o.reshape(_NT, 128),
      valid_o.reshape(_NT, 128))


def _flat(p):
    return p.reshape(_HH, _HP, 16)[:, :_WW, :9].reshape(-1)


def kernel(x, img_size, W1, b1, Wloc, bloc, Wscore, bscore):
    n, _, hh, ww = x.shape
    anchor = jnp.asarray(_anchors_np(hh, ww))
    (dy, dx, dh, dw, s0, s1, sc, y1, x1, y2, x2) = _run_head(
        x, W1, b1, Wloc, bloc, Wscore, bscore, img_size)
    rpn_locs = jnp.stack(
        [_flat(dy), _flat(dx), _flat(dh), _flat(dw)], axis=1).reshape(1, -1, 4)
    rpn_scores = jnp.stack([_flat(s0), _flat(s1)], axis=1).reshape(1, -1, 2)
    scf = _flat(sc)
    vals, order = jax.lax.top_k(scf, _N_PRE)
    pad = jnp.zeros((_NSORT - _N_PRE,), jnp.float32)
    padi = jnp.zeros((_NSORT - _N_PRE,), jnp.int32)
    orderp = jnp.concatenate([order, padi])
    y1o = jnp.concatenate([_flat(y1)[order], pad])
    x1o = jnp.concatenate([_flat(x1)[order], pad])
    y2o = jnp.concatenate([_flat(y2)[order], pad])
    x2o = jnp.concatenate([_flat(x2)[order], pad])
    valid_o = jnp.concatenate(
        [jnp.isfinite(vals).astype(jnp.float32), pad])
    del orderp
    keep = _run_nms(y1o, x1o, y2o, x2o, valid_o).reshape(-1)
    idx = jnp.nonzero(keep > 0.0, size=_N_POST, fill_value=0)[0]
    rois = jnp.stack([y1o[idx], x1o[idx], y2o[idx], x2o[idx]], axis=1)
    roi_indices = jnp.zeros((_N_POST,), dtype=jnp.int32)
    return (rpn_locs, rpn_scores, rois, roi_indices, anchor)
